# Initial kernel scaffold; baseline (speedup 1.0000x reference)
import jax
import jax.numpy as jnp
from jax.experimental import pallas as pl


def kernel(sampled_vertices, simplified_faces):
    raise NotImplementedError("write your pallas kernel here")


# speedup vs baseline: 7.3085x; 7.3085x over previous
import functools

import jax
import jax.numpy as jnp
from jax import lax
from jax.experimental import pallas as pl
from jax.experimental.pallas import tpu as pltpu
from jax.experimental.pallas import tpu_sc as plsc

_NUM_SAMPLES = 10
_K = 5
_V = 5000
_F = 2000
_FP = 2048
_N = _F * _NUM_SAMPLES
_NP = _FP * _NUM_SAMPLES
_NW = 32
_FW = _FP // _NW
_PW = _N // _NW
_CHUNK = 125
_NCHUNK = (_PW * _K) // _CHUNK

@functools.cache
def _mesh():
    return plsc.VectorSubcoreMesh(core_axis_name="c", subcore_axis_name="s",
                                  num_cores=2, num_subcores=16)


def _iota16():
    return lax.broadcasted_iota(jnp.int32, (16,), 0)


def _full16(val):
    return jnp.full((16,), val, jnp.int32)


def _sqrt16(q):
    qi = plsc.bitcast(q, jnp.int32)
    yi = (qi >> 1) + jnp.int32(0x1FBD1DF5)
    y = plsc.bitcast(yi, jnp.float32)
    for _ in range(3):
        y = 0.5 * (y + q / y)
    return jnp.where(q < 1e-30, jnp.zeros((16,), jnp.float32), y)


def _stage1_body(vtx_hbm, f0_hbm, f1_hbm, f2_hbm, wu_hbm, wv_hbm, w0_hbm,
                 a_hbm, b_hbm, ft_hbm,
                 i0_v, i1_v, i2_v, r0_v, r1_v, r2_v, wu_v, wv_v, w0_v,
                 a_loc, b_loc, ft_loc, sem):
    wid = lax.axis_index("s") * 2 + lax.axis_index("c")
    basef = wid * _FW

    pltpu.sync_copy(f0_hbm.at[pl.ds(basef, _FW)], i0_v)
    pltpu.sync_copy(f1_hbm.at[pl.ds(basef, _FW)], i1_v)
    pltpu.sync_copy(f2_hbm.at[pl.ds(basef, _FW)], i2_v)
    pltpu.sync_copy(wu_hbm.at[pl.ds(basef * _NUM_SAMPLES, _FW * _NUM_SAMPLES)], wu_v)
    pltpu.sync_copy(wv_hbm.at[pl.ds(basef * _NUM_SAMPLES, _FW * _NUM_SAMPLES)], wv_v)
    pltpu.sync_copy(w0_hbm.at[pl.ds(basef * _NUM_SAMPLES, _FW * _NUM_SAMPLES)], w0_v)

    c0 = pltpu.async_copy(vtx_hbm.at[i0_v], r0_v, sem)
    c1 = pltpu.async_copy(vtx_hbm.at[i1_v], r1_v, sem)
    c2 = pltpu.async_copy(vtx_hbm.at[i2_v], r2_v, sem)
    c0.wait()
    c1.wait()
    c2.wait()

    iota = _iota16()
    fzero = jnp.zeros((16,), jnp.float32)
    fone = jnp.ones((16,), jnp.float32)

    for g in range(_FW // 16):
        rows = g * 16 + iota

        def col(ref, c):
            return plsc.load_gather(ref, [rows, _full16(c)])

        v0x, v0y, v0z = col(r0_v, 0), col(r0_v, 1), col(r0_v, 2)
        v1x, v1y, v1z = col(r1_v, 0), col(r1_v, 1), col(r1_v, 2)
        v2x, v2y, v2z = col(r2_v, 0), col(r2_v, 1), col(r2_v, 2)

        def bput(c, val):
            plsc.store_scatter(b_loc, [rows, _full16(c)], val)

        cx = (v0x + v1x + v2x) * (1.0 / 3.0)
        cy = (v0y + v1y + v2y) * (1.0 / 3.0)
        cz = (v0z + v1z + v2z) * (1.0 / 3.0)
        cn = cx * cx + cy * cy + cz * cz
        valid = (basef + rows) < _F
        cn = jnp.where(valid, cn, jnp.full((16,), 1e30, jnp.float32))
        bput(0, -2.0 * cx)
        bput(1, -2.0 * cy)
        bput(2, -2.0 * cz)
        bput(3, cn)
        bput(4, fone)
        bput(5, fzero)
        bput(6, fzero)
        bput(7, fzero)

        e1x, e1y, e1z = v2x - v1x, v2y - v1y, v2z - v1z
        e2x, e2y, e2z = v0x - v2x, v0y - v2y, v0z - v2z
        e3x, e3y, e3z = v1x - v0x, v1y - v0y, v1z - v0z
        dx, dy, dz = v2x - v0x, v2y - v0y, v2z - v0z
        cpx = e3y * dz - e3z * dy
        cpy = e3z * dx - e3x * dz
        cpz = e3x * dy - e3y * dx

        def ftput(c, val):
            plsc.store_scatter(ft_loc, [rows, _full16(c)], val)

        def edge(ex, ey, ez, ax, ay, az, base):
            gx = cpy * ez - cpz * ey
            gy = cpz * ex - cpx * ez
            gz = cpx * ey - cpy * ex
            tx = ey * az - ez * ay
            ty = ez * ax - ex * az
            tz = ex * ay - ey * ax
            cc = -(tx * cpx + ty * cpy + tz * cpz)
            ftput(base + 0, gx)
            ftput(base + 1, gy)
            ftput(base + 2, gz)
            ftput(base + 3, cc)

        edge(e1x, e1y, e1z, v1x, v1y, v1z, 0)
        edge(e2x, e2y, e2z, v2x, v2y, v2z, 4)
        edge(e3x, e3y, e3z, v0x, v0y, v0z, 8)
        qq = cpx * cpx + cpy * cpy + cpz * cpz
        ftput(12, 0.5 * _sqrt16(qq))
        ftput(13, fzero)
        ftput(14, fzero)
        ftput(15, fzero)

        for s in range(_NUM_SAMPLES):
            widx = rows * _NUM_SAMPLES + s
            wu = plsc.load_gather(wu_v, [widx])
            wv = plsc.load_gather(wv_v, [widx])
            w0 = plsc.load_gather(w0_v, [widx])
            px = v0x * w0 + v1x * wu + v2x * wv
            py = v0y * w0 + v1y * wu + v2y * wv
            pz = v0z * w0 + v1z * wu + v2z * wv
            pn = px * px + py * py + pz * pz
            arow = rows * _NUM_SAMPLES + s

            def aput(c, val):
                plsc.store_scatter(a_loc, [arow, _full16(c)], val)

            aput(0, px)
            aput(1, py)
            aput(2, pz)
            aput(3, fone)
            aput(4, pn)
            aput(5, fzero)
            aput(6, fzero)
            aput(7, fzero)

    pltpu.sync_copy(a_loc,
                    a_hbm.at[pl.ds(basef * _NUM_SAMPLES, _FW * _NUM_SAMPLES)])
    pltpu.sync_copy(b_loc, b_hbm.at[pl.ds(basef, _FW)])
    pltpu.sync_copy(ft_loc, ft_hbm.at[pl.ds(basef, _FW)])


@functools.cache
def _stage1():
    return pl.kernel(
        _stage1_body,
        out_type=(
            jax.ShapeDtypeStruct((_NP, 8), jnp.float32),
            jax.ShapeDtypeStruct((_FP, 8), jnp.float32),
            jax.ShapeDtypeStruct((_FP, 16), jnp.float32),
        ),
        mesh=_mesh(),
        compiler_params=pltpu.CompilerParams(use_tc_tiling_on_sc=False, needs_layout_passes=False),
        scratch_types=(
        pltpu.VMEM((_FW,), jnp.int32),
        pltpu.VMEM((_FW,), jnp.int32),
        pltpu.VMEM((_FW,), jnp.int32),
        pltpu.VMEM((_FW, 16), jnp.float32),
        pltpu.VMEM((_FW, 16), jnp.float32),
        pltpu.VMEM((_FW, 16), jnp.float32),
        pltpu.VMEM((_FW * _NUM_SAMPLES,), jnp.float32),
        pltpu.VMEM((_FW * _NUM_SAMPLES,), jnp.float32),
        pltpu.VMEM((_FW * _NUM_SAMPLES,), jnp.float32),
            pltpu.VMEM((_FW * _NUM_SAMPLES, 8), jnp.float32),
            pltpu.VMEM((_FW, 8), jnp.float32),
            pltpu.VMEM((_FW, 16), jnp.float32),
            pltpu.SemaphoreType.DMA,
        ),
    )


_TOPK_ROWS = 256


def _topk_body(a_ref, b_ref, o_ref):
    a = a_ref[...]
    b = b_ref[...]
    d2 = lax.dot_general(a, b, (((1,), (1,)), ((), ())),
                         preferred_element_type=jnp.float32)
    colid = lax.broadcasted_iota(jnp.int32, (_TOPK_ROWS, _FP), 1)
    big = jnp.int32(2**30)
    cols = []
    for _ in range(_K):
        m = jnp.min(d2, axis=1, keepdims=True)
        idx = jnp.min(jnp.where(d2 == m, colid, big), axis=1)
        cols.append(idx)
        d2 = jnp.where(colid == idx[:, None], jnp.float32(jnp.inf), d2)
    zero = jnp.zeros_like(cols[0])
    o_ref[...] = jnp.stack(cols + [zero, zero, zero], axis=1)


def _topk(a, b):
    return pl.pallas_call(
        _topk_body,
        grid=(_NP // _TOPK_ROWS,),
        in_specs=[
            pl.BlockSpec((_TOPK_ROWS, 8), lambda i: (i, 0)),
            pl.BlockSpec((_FP, 8), lambda i: (0, 0)),
        ],
        out_specs=pl.BlockSpec((_TOPK_ROWS, 8), lambda i: (i, 0)),
        out_shape=jax.ShapeDtypeStruct((_NP, 8), jnp.int32),
    )(a, b)


def _stage3_body(a_hbm, ft_hbm, nn_hbm, part_hbm,
                 pts_v, nn_v, pair_v, rows_v, acc_v, sem):
    wid = lax.axis_index("s") * 2 + lax.axis_index("c")
    basep = wid * _PW

    pltpu.sync_copy(a_hbm.at[pl.ds(basep, _PW)], pts_v)
    pltpu.sync_copy(nn_hbm.at[pl.ds(basep, _PW)], nn_v)

    iota = _iota16()
    for c in range(_NCHUNK):
        for g in range(8):
            r = g * 16 + iota
            j = c * _CHUNK + r
            i_raw = j // _K
            kk = j - i_raw * _K
            i = jnp.minimum(i_raw, _PW - 1)
            val = plsc.load_gather(nn_v, [i, kk])
            pair_v[c, pl.ds(g * 16, 16)] = val

    copies = [pltpu.async_copy(ft_hbm.at[pair_v.at[c]], rows_v.at[c], sem)
              for c in range(_NCHUNK)]

    def chunk(c, acc):
        cvec = _full16(0) + c
        for g in range(8):
            r = g * 16 + iota
            j = c * _CHUNK + r
            i_raw = j // _K
            i = jnp.minimum(i_raw, _PW - 1)
            px = plsc.load_gather(pts_v, [i, _full16(0)])
            py = plsc.load_gather(pts_v, [i, _full16(1)])
            pz = plsc.load_gather(pts_v, [i, _full16(2)])

            def t(cc):
                return plsc.load_gather(rows_v, [cvec, r, _full16(cc)])

            uu = px * t(0) + py * t(1) + pz * t(2) + t(3)
            vv = px * t(4) + py * t(5) + pz * t(6) + t(7)
            ww = px * t(8) + py * t(9) + pz * t(10) + t(11)
            area = t(12)
            inside = ((area > 1e-6) & (uu >= 0.0) & (vv >= 0.0)
                      & (ww >= 0.0) & (r < _CHUNK))
            acc = acc + jnp.where(inside, area, jnp.zeros((16,), jnp.float32))
        return acc

    for cp in copies:
        cp.wait()
    acc = lax.fori_loop(0, _NCHUNK, chunk, jnp.zeros((16,), jnp.float32))
    acc_v[0, :] = acc
    pltpu.sync_copy(acc_v, part_hbm.at[pl.ds(wid, 1)])


@functools.cache
def _stage3():
    return pl.kernel(
        _stage3_body,
        out_type=jax.ShapeDtypeStruct((_NW, 16), jnp.float32),
        mesh=_mesh(),
        compiler_params=pltpu.CompilerParams(use_tc_tiling_on_sc=False, needs_layout_passes=False),
        scratch_types=(
            pltpu.VMEM((_PW, 8), jnp.float32),
            pltpu.VMEM((_PW, 8), jnp.int32),
            pltpu.VMEM((_NCHUNK, 128), jnp.int32),
            pltpu.VMEM((_NCHUNK, 128, 16), jnp.float32),
            pltpu.VMEM((1, 16), jnp.float32),
            pltpu.SemaphoreType.DMA,
        ),
    )


def _barycentric_weights():
    key = jax.random.key(42)
    ku, kv = jax.random.split(key)
    u = jax.random.uniform(ku, (_F, _NUM_SAMPLES), dtype=jnp.float32)
    v = jax.random.uniform(kv, (_F, _NUM_SAMPLES), dtype=jnp.float32)
    cond = (u + v) > 1.0
    u = jnp.where(cond, 1.0 - u, u)
    v = jnp.where(cond, 1.0 - v, v)
    return u, v


def kernel(sampled_vertices, simplified_faces):
    vtx = jnp.pad(sampled_vertices.astype(jnp.float32), ((0, 0), (0, 13)))
    faces = simplified_faces.astype(jnp.int32)
    faces = jnp.pad(faces, ((0, _FP - _F), (0, 0)))
    u, v = _barycentric_weights()
    pad = ((0, _FP - _F), (0, 0))
    wu = jnp.pad(u, pad).reshape(-1)
    wv = jnp.pad(v, pad).reshape(-1)
    w0 = jnp.pad(1.0 - u - v, pad).reshape(-1)

    a, b, ft = _stage1()(vtx, faces[:, 0], faces[:, 1], faces[:, 2], wu, wv, w0)
    nn = _topk(a, b)
    part = _stage3()(a, ft, nn)
    return jnp.sum(part)
